# baseline (speedup 1.0000x reference)
import functools

import jax
import jax.numpy as jnp
from jax import lax
from jax.experimental import pallas as pl
from jax.experimental.pallas import tpu as pltpu
from jax.experimental.pallas import tpu_sc as plsc

BATCH = 16384
NUM_FIELDS = 26
FIELD_DIM = 40000
TOTAL_VOCAB = NUM_FIELDS * FIELD_DIM
TDIM = 16

NC, NS, LANES = 2, 16, 16
NW = NC * NS
BPW = BATCH // NW
NSTAGE = 13
TSLICE = TOTAL_VOCAB // NSTAGE

_mesh = plsc.VectorSubcoreMesh(
    core_axis_name="c", subcore_axis_name="s", num_cores=NC, num_subcores=NS
)


@functools.partial(
    pl.kernel,
    out_type=jax.ShapeDtypeStruct((BATCH,), jnp.float32),
    mesh=_mesh,
    compiler_params=pltpu.CompilerParams(use_tc_tiling_on_sc=True),
    scratch_types=[
        pltpu.VMEM((NUM_FIELDS * BPW,), jnp.int32),
        pltpu.VMEM((NUM_FIELDS * BPW,), jnp.float32),
        pltpu.VMEM((TDIM * BPW,), jnp.float32),
        pltpu.VMEM((TDIM * LANES + LANES,), jnp.float32),
        pltpu.VMEM((BPW,), jnp.float32),
        pltpu.VMEM_SHARED((TOTAL_VOCAB,), jnp.float32),
        pltpu.SemaphoreType.DMA,
    ],
)
def _fl_kernel(xiT_hbm, tT_hbm, tab_hbm, pv_hbm, out_hbm,
               idx_v, vals_v, t1_v, pv_v, out_v, tab_sh, sem):
    c = lax.axis_index("c")
    s = lax.axis_index("s")
    wid = s * NC + c

    @pl.when(s < NSTAGE)
    def _stage():
        pltpu.sync_copy(tab_hbm.at[pl.ds(s * TSLICE, TSLICE)],
                        tab_sh.at[pl.ds(s * TSLICE, TSLICE)])

    col = pl.ds(wid * BPW, BPW)
    for f in range(NUM_FIELDS):
        pltpu.sync_copy(xiT_hbm.at[f, col], idx_v.at[pl.ds(f * BPW, BPW)])
    for k in range(TDIM):
        pltpu.sync_copy(tT_hbm.at[k, col], t1_v.at[pl.ds(k * BPW, BPW)])
    pltpu.sync_copy(pv_hbm, pv_v)

    plsc.subcore_barrier()

    for half in range(2):
        copies = [
            pltpu.make_async_copy(
                tab_sh.at[idx_v.at[pl.ds(f * BPW, BPW)]],
                vals_v.at[pl.ds(f * BPW, BPW)], sem)
            for f in range(half * 13, half * 13 + 13)
        ]
        for cp in copies:
            cp.start()
        for cp in copies:
            cp.wait()

    c0 = pv_v[pl.ds(TDIM * LANES, LANES)]
    for j in range(BPW // LANES):
        acc = c0
        sl = pl.ds(j * LANES, LANES)
        for f in range(NUM_FIELDS):
            acc = acc + vals_v[pl.ds(f * BPW + j * LANES, LANES)]
        for k in range(TDIM):
            acc = acc + pv_v[pl.ds(k * LANES, LANES)] * t1_v[pl.ds(k * BPW + j * LANES, LANES)]
        out_v[sl] = acc
    pltpu.sync_copy(out_v, out_hbm.at[pl.ds(wid * BPW, BPW)])


def kernel(x, t, fc_table, lin_W, lin_b, bias):
    offsets = jnp.arange(NUM_FIELDS, dtype=x.dtype) * FIELD_DIM
    xiT = (x + offsets[None, :]).T
    tT = t.T
    tab = fc_table.reshape(TOTAL_VOCAB)
    pv = jnp.concatenate([
        jnp.repeat(lin_W.reshape(TDIM), LANES),
        jnp.broadcast_to((lin_b + bias).reshape(1), (LANES,)),
    ]).astype(jnp.float32)
    out = _fl_kernel(xiT, tT, tab, pv)
    return out.reshape(BATCH, 1)
